# baseline (speedup 1.0000x reference)
import functools

import jax
import jax.numpy as jnp
from jax import lax
from jax.experimental import pallas as pl
from jax.experimental.pallas import tpu as pltpu
from jax.experimental.pallas import tpu_sc as plsc

E = 8
TOPK = 2
D = 1024
F = 2048
T = 2048
P = T * TOPK
BM = 256
NPAD = 5888
NBLK = NPAD // BM
NC, NS, L = 2, 16, 16
NW = NC * NS
PPW = P // NW
CHW = PPW // L
NCH = P // L


def _router_body(x_ref, gw_ref, w_ref, i_ref):
    x = x_ref[...]
    gw = gw_ref[...]
    logits = lax.dot_general(
        x, gw, (((1,), (1,)), ((), ())),
        preferred_element_type=jnp.float32,
    )
    m = jnp.max(logits, axis=1, keepdims=True)
    ex = jnp.exp(logits - m)
    probs = ex / jnp.sum(ex, axis=1, keepdims=True)
    iota = lax.broadcasted_iota(jnp.int32, (T, E), 1)
    m1 = jnp.max(probs, axis=1, keepdims=True)
    a1 = jnp.min(jnp.where(probs == m1, iota, E), axis=1, keepdims=True)
    not1 = iota != a1
    m2 = jnp.max(jnp.where(not1, probs, -1.0), axis=1, keepdims=True)
    a2 = jnp.min(jnp.where((probs == m2) & not1, iota, E), axis=1,
                 keepdims=True)
    w_ref[...] = jnp.concatenate([m1, m2], axis=1)
    i_ref[...] = jnp.concatenate([a1, a2], axis=1)


def _router(x, gate_w):
    return pl.pallas_call(
        _router_body,
        out_shape=(
            jax.ShapeDtypeStruct((T, TOPK), jnp.float32),
            jax.ShapeDtypeStruct((T, TOPK), jnp.int32),
        ),
    )(x, gate_w)


def _dispatch_body(eid_hbm, x_hbm, dest_hbm, be_hbm, ri_hbm, sched_hbm,
                   xs_hbm, eid_v, rank_v, dest_v, off_v, be_v, ri_v, sched_v,
                   rows_v, gsem0, gsem1, ssem0, ssem1):
    wid = lax.axis_index("c") * NS + lax.axis_index("s")
    pltpu.sync_copy(eid_hbm, eid_v)
    lane = lax.broadcasted_iota(jnp.int32, (L,), 0)

    def count_chunk(i, cnts):
        v = eid_v[pl.ds(i * L, L)]
        return tuple(c + jnp.sum((v == e).astype(jnp.int32))
                     for e, c in enumerate(cnts))

    zero = jnp.int32(0)
    my_lo = wid * CHW
    cnts = lax.fori_loop(0, my_lo, count_chunk, (zero,) * E)
    for j in range(CHW):
        v = eid_v[pl.ds((my_lo + j) * L, L)]
        rank = jnp.zeros((L,), jnp.int32)
        nxt = []
        for e in range(E):
            mk = v == e
            mi = mk.astype(jnp.int32)
            cs = plsc.cumsum(mi)
            rank = rank + jnp.where(mk, cs - 1 + cnts[e], 0)
            nxt.append(cnts[e] + jnp.sum(mi))
        cnts = tuple(nxt)
        rank_v[pl.ds(j * L, L)] = rank
    tot = lax.fori_loop(my_lo + CHW, NCH, count_chunk, cnts)
    off = []
    acc = zero
    for e in range(E):
        off.append(acc)
        acc = acc + ((tot[e] + (BM - 1)) >> 8 << 8)
    off_vec = jnp.zeros((L,), jnp.int32)
    for e in range(E):
        off_vec = jnp.where(lane == e, off[e], off_vec)
    off_v[...] = off_vec
    for j in range(CHW):
        v = eid_v[pl.ds((my_lo + j) * L, L)]
        base = plsc.load_gather(off_v, [v])
        dest_v[pl.ds(j * L, L)] = base + rank_v[pl.ds(j * L, L)]
    pltpu.sync_copy(dest_v, dest_hbm.at[pl.ds(wid * PPW, PPW)])

    @pl.when(wid == 0)
    def _():
        be0 = jnp.zeros((L,), jnp.int32)
        be1 = jnp.zeros((L,), jnp.int32)
        b0 = lane * BM
        b1 = (lane + L) * BM
        for e in range(1, E):
            be0 = be0 + (b0 >= off[e]).astype(jnp.int32)
            be1 = be1 + (b1 >= off[e]).astype(jnp.int32)
        tot_vec = jnp.zeros((L,), jnp.int32)
        for e in range(E):
            tot_vec = jnp.where(lane == e, tot[e], tot_vec)
        pres = (tot_vec > 0) & (lane < E)
        presi = pres.astype(jnp.int32)
        nruns = jnp.sum(presi)
        place = plsc.cumsum(presi) - 1
        sched_v[...] = jnp.full((L,), E - 1, jnp.int32)
        plsc.store_scatter(sched_v, [jnp.maximum(place, 0)], lane, mask=pres)
        pltpu.sync_copy(sched_v, sched_hbm)
        be_v[pl.ds(0, L)] = be0
        be_v[pl.ds(L, L)] = jnp.where(lane == L - 1, nruns, be1)
        pltpu.sync_copy(be_v, be_hbm)
        sh0 = plsc.load_gather(be_v, [jnp.maximum(lane - 1, 0)])
        ch0 = ((be0 != sh0) & (lane > 0)).astype(jnp.int32)
        ri0 = plsc.cumsum(ch0)
        sh1 = plsc.load_gather(be_v, [lane + (L - 1)])
        ch1 = (be1 != sh1).astype(jnp.int32)
        ri1 = plsc.cumsum(ch1) + jnp.sum(ch0)
        ri_v[pl.ds(0, L)] = ri0
        ri_v[pl.ds(L, L)] = ri1
        pltpu.sync_copy(ri_v, ri_hbm)

    def gath(j, b, sem):
        tok = ((wid * PPW + j * L) + lane) >> 1
        return pltpu.async_copy(x_hbm.at[tok], rows_v.at[b], sem)

    def scat(j, b, sem):
        d = dest_v[pl.ds(j * L, L)]
        return pltpu.async_copy(rows_v.at[b], xs_hbm.at[d], sem)

    gsem = (gsem0, gsem1)
    ssem = (ssem0, ssem1)
    gd = {0: gath(0, 0, gsem[0])}
    sd = {}
    for j in range(CHW):
        p = j % 2
        gd[p].wait()
        sd[p] = scat(j, p, ssem[p])
        if j + 1 < CHW:
            q = 1 - p
            if q in sd:
                sd[q].wait()
            gd[q] = gath(j + 1, q, gsem[q])
    for p in (0, 1):
        if p in sd:
            sd[p].wait()


def _dispatch(eid, x):
    mesh = plsc.VectorSubcoreMesh(core_axis_name="c", subcore_axis_name="s",
                                  num_cores=NC, num_subcores=NS)
    return pl.kernel(
        _dispatch_body,
        out_type=(
            jax.ShapeDtypeStruct((P,), jnp.int32),
            jax.ShapeDtypeStruct((NW,), jnp.int32),
            jax.ShapeDtypeStruct((NW,), jnp.int32),
            jax.ShapeDtypeStruct((L,), jnp.int32),
            jax.ShapeDtypeStruct((NPAD, D), jnp.float32),
        ),
        mesh=mesh,
        scratch_types=[
            pltpu.VMEM((P,), jnp.int32),
            pltpu.VMEM((PPW,), jnp.int32),
            pltpu.VMEM((PPW,), jnp.int32),
            pltpu.VMEM((L,), jnp.int32),
            pltpu.VMEM((NW,), jnp.int32),
            pltpu.VMEM((NW,), jnp.int32),
            pltpu.VMEM((L,), jnp.int32),
            pltpu.VMEM((2, L, D), jnp.float32),
            pltpu.SemaphoreType.DMA,
            pltpu.SemaphoreType.DMA,
            pltpu.SemaphoreType.DMA,
            pltpu.SemaphoreType.DMA,
        ],
        compiler_params=pltpu.CompilerParams(needs_layout_passes=False),
    )(eid, x)


def _gmm_body(be_ref, ri_ref, sd_ref, xs_ref, w1_hbm, w3_hbm, w2_hbm, out_ref,
              w1b, w3b, w2b, sems):
    b = pl.program_id(0)
    e = be_ref[b]
    r = ri_ref[b]
    nruns = be_ref[NW - 1]
    first = jnp.logical_or(b == 0, be_ref[jnp.maximum(b - 1, 0)] != e)
    slot = lax.rem(r, 2)
    nslot = lax.rem(r + 1, 2)
    nxt = sd_ref[jnp.minimum(r + 1, L - 1)]
    bufs = ((w1_hbm, w1b), (w3_hbm, w3b), (w2_hbm, w2b))

    @pl.when(b == 0)
    def _():
        for hbm, buf in bufs:
            pltpu.make_async_copy(hbm.at[e], buf.at[0], sems.at[0]).start()

    @pl.when(first & (r < nruns))
    def _():
        for hbm, buf in bufs:
            pltpu.make_async_copy(hbm.at[e], buf.at[slot],
                                  sems.at[slot]).wait()

    @pl.when(first & (r + 1 < nruns))
    def _():
        for hbm, buf in bufs:
            pltpu.make_async_copy(hbm.at[nxt], buf.at[nslot],
                                  sems.at[nslot]).start()

    x = xs_ref[...].astype(jnp.bfloat16)
    w1 = w1b[slot].astype(jnp.bfloat16)
    w3 = w3b[slot].astype(jnp.bfloat16)
    w2 = w2b[slot].astype(jnp.bfloat16)
    dn = (((1,), (1,)), ((), ()))
    g = lax.dot_general(x, w1, dn, preferred_element_type=jnp.float32)
    u = lax.dot_general(x, w3, dn, preferred_element_type=jnp.float32)
    a = (g * jax.nn.sigmoid(g) * u).astype(jnp.bfloat16)
    out_ref[...] = lax.dot_general(a, w2, dn,
                                   preferred_element_type=jnp.float32)


def _gmm(be, ri, sched, xs, w1, w3, w2):
    grid_spec = pltpu.PrefetchScalarGridSpec(
        num_scalar_prefetch=3,
        grid=(NBLK,),
        in_specs=[
            pl.BlockSpec((BM, D), lambda b, be, ri, sd: (b, 0)),
            pl.BlockSpec(memory_space=pltpu.MemorySpace.HBM),
            pl.BlockSpec(memory_space=pltpu.MemorySpace.HBM),
            pl.BlockSpec(memory_space=pltpu.MemorySpace.HBM),
        ],
        out_specs=pl.BlockSpec((BM, D), lambda b, be, ri, sd: (b, 0)),
        scratch_shapes=[
            pltpu.VMEM((2, F, D), jnp.float32),
            pltpu.VMEM((2, F, D), jnp.float32),
            pltpu.VMEM((2, D, F), jnp.float32),
            pltpu.SemaphoreType.DMA((2,)),
        ],
    )
    return pl.pallas_call(
        _gmm_body,
        grid_spec=grid_spec,
        out_shape=jax.ShapeDtypeStruct((NPAD, D), jnp.float32),
        compiler_params=pltpu.CompilerParams(
            vmem_limit_bytes=120 * 1024 * 1024,
        ),
    )(be, ri, sched, xs, w1, w3, w2)


def _combine_body(hs_hbm, dest_hbm, wts_hbm, out_hbm,
                  dest_v, wts_v, rows_v, out_v, gsem0, gsem1):
    wid = lax.axis_index("c") * NS + lax.axis_index("s")
    pltpu.sync_copy(dest_hbm.at[pl.ds(wid * PPW, PPW)], dest_v)
    pltpu.sync_copy(wts_hbm.at[pl.ds(wid * PPW, PPW)], wts_v.at[pl.ds(L, PPW)])
    tpw = PPW // TOPK

    def gath(j, b, sem):
        d = dest_v[pl.ds(j * L, L)]
        return pltpu.async_copy(hs_hbm.at[d], rows_v.at[b], sem)

    gsem = (gsem0, gsem1)
    gd = {0: gath(0, 0, gsem[0])}
    for j in range(CHW):
        p = j % 2
        if j + 1 < CHW:
            gd[1 - p] = gath(j + 1, 1 - p, gsem[1 - p])
        gd[p].wait()
        wv = [plsc.load_gather(wts_v,
                               [jnp.full((L,), L + j * L + t, jnp.int32)])
              for t in range(L)]

        def col(k):
            for i in range(L // TOPK):
                a = rows_v[p, 2 * i, pl.ds(k * L, L)]
                b = rows_v[p, 2 * i + 1, pl.ds(k * L, L)]
                out_v[i, pl.ds(k * L, L)] = wv[2 * i] * a + wv[2 * i + 1] * b

        plsc.parallel_loop(0, D // L, unroll=2)(col)
        pltpu.sync_copy(
            out_v, out_hbm.at[pl.ds(wid * tpw + j * (L // TOPK), L // TOPK)])


def _combine(hs, dest, wts):
    mesh = plsc.VectorSubcoreMesh(core_axis_name="c", subcore_axis_name="s",
                                  num_cores=NC, num_subcores=NS)
    return pl.kernel(
        _combine_body,
        out_type=jax.ShapeDtypeStruct((T, D), jnp.float32),
        mesh=mesh,
        scratch_types=[
            pltpu.VMEM((PPW,), jnp.int32),
            pltpu.VMEM((PPW + L,), jnp.float32),
            pltpu.VMEM((2, L, D), jnp.float32),
            pltpu.VMEM((L // TOPK, D), jnp.float32),
            pltpu.SemaphoreType.DMA,
            pltpu.SemaphoreType.DMA,
        ],
        compiler_params=pltpu.CompilerParams(needs_layout_passes=False),
    )(hs, dest, wts)


def kernel(hidden_states, gate_w, w1, w2, w3):
    orig_shape = hidden_states.shape
    x = hidden_states.reshape(T, D)
    topk_w, topk_ids = _router(x, gate_w)
    eid = topk_ids.reshape(P)
    wts = topk_w.reshape(P)
    dest, be, ri, sched, xs = _dispatch(eid, x)
    hs = _gmm(be, ri, sched, xs, w1, w3, w2)
    out = _combine(hs, dest, wts)
    return out.reshape(orig_shape)


# speedup vs baseline: 1.0119x; 1.0119x over previous
import functools

import jax
import jax.numpy as jnp
from jax import lax
from jax.experimental import pallas as pl
from jax.experimental.pallas import tpu as pltpu
from jax.experimental.pallas import tpu_sc as plsc

E = 8
TOPK = 2
D = 1024
F = 2048
T = 2048
P = T * TOPK
BM = 256
NPAD = 5888
NBLK = NPAD // BM
NC, NS, L = 2, 16, 16
NW = NC * NS
PPW = P // NW
CHW = PPW // L
NCH = P // L


def _router_body(x_ref, gw_ref, w_ref, i_ref):
    x = x_ref[...]
    gw = gw_ref[...]
    logits = lax.dot_general(
        x, gw, (((1,), (1,)), ((), ())),
        preferred_element_type=jnp.float32,
    )
    m = jnp.max(logits, axis=1, keepdims=True)
    ex = jnp.exp(logits - m)
    probs = ex / jnp.sum(ex, axis=1, keepdims=True)
    iota = lax.broadcasted_iota(jnp.int32, (T, E), 1)
    m1 = jnp.max(probs, axis=1, keepdims=True)
    a1 = jnp.min(jnp.where(probs == m1, iota, E), axis=1, keepdims=True)
    not1 = iota != a1
    m2 = jnp.max(jnp.where(not1, probs, -1.0), axis=1, keepdims=True)
    a2 = jnp.min(jnp.where((probs == m2) & not1, iota, E), axis=1,
                 keepdims=True)
    w_ref[...] = jnp.concatenate([m1, m2], axis=1)
    i_ref[...] = jnp.concatenate([a1, a2], axis=1)


def _router(x, gate_w):
    return pl.pallas_call(
        _router_body,
        out_shape=(
            jax.ShapeDtypeStruct((T, TOPK), jnp.float32),
            jax.ShapeDtypeStruct((T, TOPK), jnp.int32),
        ),
    )(x, gate_w)


def _dispatch_body(eid_hbm, x_hbm, dest_hbm, be_hbm, ri_hbm, sched_hbm,
                   xs_hbm, eid_v, rank_v, dest_v, off_v, be_v, ri_v, sched_v,
                   rows_v, gsem0, gsem1, ssem0, ssem1):
    wid = lax.axis_index("c") * NS + lax.axis_index("s")
    pltpu.sync_copy(eid_hbm, eid_v)
    lane = lax.broadcasted_iota(jnp.int32, (L,), 0)

    def count_chunk(i, cnts):
        v = eid_v[pl.ds(i * L, L)]
        return tuple(c + jnp.sum((v == e).astype(jnp.int32))
                     for e, c in enumerate(cnts))

    zero = jnp.int32(0)
    my_lo = wid * CHW
    cnts = lax.fori_loop(0, my_lo, count_chunk, (zero,) * E)
    for j in range(CHW):
        v = eid_v[pl.ds((my_lo + j) * L, L)]
        rank = jnp.zeros((L,), jnp.int32)
        nxt = []
        for e in range(E):
            mk = v == e
            mi = mk.astype(jnp.int32)
            cs = plsc.cumsum(mi)
            rank = rank + jnp.where(mk, cs - 1 + cnts[e], 0)
            nxt.append(cnts[e] + jnp.sum(mi))
        cnts = tuple(nxt)
        rank_v[pl.ds(j * L, L)] = rank
    tot = lax.fori_loop(my_lo + CHW, NCH, count_chunk, cnts)
    off = []
    acc = zero
    for e in range(E):
        off.append(acc)
        acc = acc + ((tot[e] + (BM - 1)) >> 8 << 8)
    off_vec = jnp.zeros((L,), jnp.int32)
    for e in range(E):
        off_vec = jnp.where(lane == e, off[e], off_vec)
    off_v[...] = off_vec
    for j in range(CHW):
        v = eid_v[pl.ds((my_lo + j) * L, L)]
        base = plsc.load_gather(off_v, [v])
        dest_v[pl.ds(j * L, L)] = base + rank_v[pl.ds(j * L, L)]
    pltpu.sync_copy(dest_v, dest_hbm.at[pl.ds(wid * PPW, PPW)])

    @pl.when(wid == 0)
    def _():
        be0 = jnp.zeros((L,), jnp.int32)
        be1 = jnp.zeros((L,), jnp.int32)
        b0 = lane * BM
        b1 = (lane + L) * BM
        for e in range(1, E):
            be0 = be0 + (b0 >= off[e]).astype(jnp.int32)
            be1 = be1 + (b1 >= off[e]).astype(jnp.int32)
        tot_vec = jnp.zeros((L,), jnp.int32)
        for e in range(E):
            tot_vec = jnp.where(lane == e, tot[e], tot_vec)
        pres = (tot_vec > 0) & (lane < E)
        presi = pres.astype(jnp.int32)
        nruns = jnp.sum(presi)
        place = plsc.cumsum(presi) - 1
        sched_v[...] = jnp.full((L,), E - 1, jnp.int32)
        plsc.store_scatter(sched_v, [jnp.maximum(place, 0)], lane, mask=pres)
        pltpu.sync_copy(sched_v, sched_hbm)
        be_v[pl.ds(0, L)] = be0
        be_v[pl.ds(L, L)] = jnp.where(lane == L - 1, nruns, be1)
        pltpu.sync_copy(be_v, be_hbm)
        sh0 = plsc.load_gather(be_v, [jnp.maximum(lane - 1, 0)])
        ch0 = ((be0 != sh0) & (lane > 0)).astype(jnp.int32)
        ri0 = plsc.cumsum(ch0)
        sh1 = plsc.load_gather(be_v, [lane + (L - 1)])
        ch1 = (be1 != sh1).astype(jnp.int32)
        ri1 = plsc.cumsum(ch1) + jnp.sum(ch0)
        ri_v[pl.ds(0, L)] = ri0
        ri_v[pl.ds(L, L)] = ri1
        pltpu.sync_copy(ri_v, ri_hbm)

    def gath(j, b, sem):
        tok = ((wid * PPW + j * L) + lane) >> 1
        return pltpu.async_copy(x_hbm.at[tok], rows_v.at[b], sem)

    def scat(j, b, sem):
        d = dest_v[pl.ds(j * L, L)]
        return pltpu.async_copy(rows_v.at[b], xs_hbm.at[d], sem)

    gsem = (gsem0, gsem1)
    ssem = (ssem0, ssem1)
    gd = {0: gath(0, 0, gsem[0])}
    sd = {}
    for j in range(CHW):
        p = j % 2
        gd[p].wait()
        sd[p] = scat(j, p, ssem[p])
        if j + 1 < CHW:
            q = 1 - p
            if q in sd:
                sd[q].wait()
            gd[q] = gath(j + 1, q, gsem[q])
    for p in (0, 1):
        if p in sd:
            sd[p].wait()


def _dispatch(eid, x):
    mesh = plsc.VectorSubcoreMesh(core_axis_name="c", subcore_axis_name="s",
                                  num_cores=NC, num_subcores=NS)
    return pl.kernel(
        _dispatch_body,
        out_type=(
            jax.ShapeDtypeStruct((P,), jnp.int32),
            jax.ShapeDtypeStruct((NW,), jnp.int32),
            jax.ShapeDtypeStruct((NW,), jnp.int32),
            jax.ShapeDtypeStruct((L,), jnp.int32),
            jax.ShapeDtypeStruct((NPAD, D), jnp.float32),
        ),
        mesh=mesh,
        scratch_types=[
            pltpu.VMEM((P,), jnp.int32),
            pltpu.VMEM((PPW,), jnp.int32),
            pltpu.VMEM((PPW,), jnp.int32),
            pltpu.VMEM((L,), jnp.int32),
            pltpu.VMEM((NW,), jnp.int32),
            pltpu.VMEM((NW,), jnp.int32),
            pltpu.VMEM((L,), jnp.int32),
            pltpu.VMEM((2, L, D), jnp.float32),
            pltpu.SemaphoreType.DMA,
            pltpu.SemaphoreType.DMA,
            pltpu.SemaphoreType.DMA,
            pltpu.SemaphoreType.DMA,
        ],
        compiler_params=pltpu.CompilerParams(needs_layout_passes=False),
    )(eid, x)


def _gmm_body(be_ref, ri_ref, sd_ref, xs_ref, w1_hbm, w3_hbm, w2_hbm, out_ref,
              w1b, w3b, w2b, sems):
    b = pl.program_id(0)
    e = be_ref[b]
    r = ri_ref[b]
    nruns = be_ref[NW - 1]
    first = jnp.logical_or(b == 0, be_ref[jnp.maximum(b - 1, 0)] != e)
    slot = lax.rem(r, 2)
    nslot = lax.rem(r + 1, 2)
    nxt = sd_ref[jnp.minimum(r + 1, L - 1)]
    bufs = ((w1_hbm, w1b), (w3_hbm, w3b), (w2_hbm, w2b))

    @pl.when(b == 0)
    def _():
        for hbm, buf in bufs:
            pltpu.make_async_copy(hbm.at[e], buf.at[0], sems.at[0]).start()

    @pl.when(first & (r < nruns))
    def _():
        for hbm, buf in bufs:
            pltpu.make_async_copy(hbm.at[e], buf.at[slot],
                                  sems.at[slot]).wait()

    @pl.when(first & (r + 1 < nruns))
    def _():
        for hbm, buf in bufs:
            pltpu.make_async_copy(hbm.at[nxt], buf.at[nslot],
                                  sems.at[nslot]).start()

    x = xs_ref[...].astype(jnp.bfloat16)
    w1 = w1b[slot].astype(jnp.bfloat16)
    w3 = w3b[slot].astype(jnp.bfloat16)
    w2 = w2b[slot].astype(jnp.bfloat16)
    dn = (((1,), (1,)), ((), ()))
    g = lax.dot_general(x, w1, dn, preferred_element_type=jnp.float32)
    u = lax.dot_general(x, w3, dn, preferred_element_type=jnp.float32)
    a = (g * jax.nn.sigmoid(g) * u).astype(jnp.bfloat16)
    out_ref[...] = lax.dot_general(a, w2, dn,
                                   preferred_element_type=jnp.float32)


def _gmm(be, ri, sched, xs, w1, w3, w2):
    grid_spec = pltpu.PrefetchScalarGridSpec(
        num_scalar_prefetch=3,
        grid=(NBLK,),
        in_specs=[
            pl.BlockSpec((BM, D), lambda b, be, ri, sd: (b, 0)),
            pl.BlockSpec(memory_space=pltpu.MemorySpace.HBM),
            pl.BlockSpec(memory_space=pltpu.MemorySpace.HBM),
            pl.BlockSpec(memory_space=pltpu.MemorySpace.HBM),
        ],
        out_specs=pl.BlockSpec((BM, D), lambda b, be, ri, sd: (b, 0)),
        scratch_shapes=[
            pltpu.VMEM((2, F, D), jnp.float32),
            pltpu.VMEM((2, F, D), jnp.float32),
            pltpu.VMEM((2, D, F), jnp.float32),
            pltpu.SemaphoreType.DMA((2,)),
        ],
    )
    return pl.pallas_call(
        _gmm_body,
        grid_spec=grid_spec,
        out_shape=jax.ShapeDtypeStruct((NPAD, D), jnp.float32),
        compiler_params=pltpu.CompilerParams(
            vmem_limit_bytes=120 * 1024 * 1024,
        ),
    )(be, ri, sched, xs, w1, w3, w2)


def _combine_body(hs_hbm, dest_hbm, wts_hbm, out_hbm,
                  dest_v, wts_v, rows_v, out_v, gsem0, gsem1, osem0, osem1):
    wid = lax.axis_index("c") * NS + lax.axis_index("s")
    pltpu.sync_copy(dest_hbm.at[pl.ds(wid * PPW, PPW)], dest_v)
    pltpu.sync_copy(wts_hbm.at[pl.ds(wid * PPW, PPW)], wts_v.at[pl.ds(L, PPW)])
    tpw = PPW // TOPK

    def gath(j, b, sem):
        d = dest_v[pl.ds(j * L, L)]
        return pltpu.async_copy(hs_hbm.at[d], rows_v.at[b], sem)

    gsem = (gsem0, gsem1)
    osem = (osem0, osem1)
    gd = {0: gath(0, 0, gsem[0])}
    od = {}
    for j in range(CHW):
        p = j % 2
        if j + 1 < CHW:
            gd[1 - p] = gath(j + 1, 1 - p, gsem[1 - p])
        gd[p].wait()
        if p in od:
            od[p].wait()
        wv = [plsc.load_gather(wts_v,
                               [jnp.full((L,), L + j * L + t, jnp.int32)])
              for t in range(L)]

        def col(k):
            for i in range(L // TOPK):
                a = rows_v[p, 2 * i, pl.ds(k * L, L)]
                b = rows_v[p, 2 * i + 1, pl.ds(k * L, L)]
                out_v[p, i, pl.ds(k * L, L)] = (wv[2 * i] * a
                                                + wv[2 * i + 1] * b)

        plsc.parallel_loop(0, D // L, unroll=4)(col)
        od[p] = pltpu.async_copy(
            out_v.at[p],
            out_hbm.at[pl.ds(wid * tpw + j * (L // TOPK), L // TOPK)],
            osem[p])
    for p in (0, 1):
        if p in od:
            od[p].wait()


def _combine(hs, dest, wts):
    mesh = plsc.VectorSubcoreMesh(core_axis_name="c", subcore_axis_name="s",
                                  num_cores=NC, num_subcores=NS)
    return pl.kernel(
        _combine_body,
        out_type=jax.ShapeDtypeStruct((T, D), jnp.float32),
        mesh=mesh,
        scratch_types=[
            pltpu.VMEM((PPW,), jnp.int32),
            pltpu.VMEM((PPW + L,), jnp.float32),
            pltpu.VMEM((2, L, D), jnp.float32),
            pltpu.VMEM((2, L // TOPK, D), jnp.float32),
            pltpu.SemaphoreType.DMA,
            pltpu.SemaphoreType.DMA,
            pltpu.SemaphoreType.DMA,
            pltpu.SemaphoreType.DMA,
        ],
        compiler_params=pltpu.CompilerParams(needs_layout_passes=False),
    )(hs, dest, wts)


def kernel(hidden_states, gate_w, w1, w2, w3):
    orig_shape = hidden_states.shape
    x = hidden_states.reshape(T, D)
    topk_w, topk_ids = _router(x, gate_w)
    eid = topk_ids.reshape(P)
    wts = topk_w.reshape(P)
    dest, be, ri, sched, xs = _dispatch(eid, x)
    hs = _gmm(be, ri, sched, xs, w1, w3, w2)
    out = _combine(hs, dest, wts)
    return out.reshape(orig_shape)
